# baseline (speedup 1.0000x reference)
import functools

import jax
import jax.numpy as jnp
from jax import lax
from jax.experimental import pallas as pl
from jax.experimental.pallas import tpu as pltpu
from jax.experimental.pallas import tpu_sc as plsc

H = 128
G = 128
NC = 2
NS = 16
NW = NC * NS
NPAD = 10240


def _sc_mesh():
    return plsc.VectorSubcoreMesh(core_axis_name="c", subcore_axis_name="s")


def _sc_gather(x, dst, src):
    E = dst.shape[0]
    n_groups = E // G
    n_iters = (n_groups + NW - 1) // NW

    @functools.partial(
        pl.kernel,
        mesh=_sc_mesh(),
        out_type=(
            jax.ShapeDtypeStruct((E, H), jnp.float32),
            jax.ShapeDtypeStruct((E, H), jnp.float32),
        ),
        scratch_types=[
            pltpu.VMEM((G,), jnp.int32),
            pltpu.VMEM((G, H), jnp.float32),
            pltpu.VMEM((G,), jnp.int32),
            pltpu.VMEM((G, H), jnp.float32),
            pltpu.SemaphoreType.DMA,
            pltpu.SemaphoreType.DMA,
        ],
    )
    def k(x_hbm, dst_hbm, src_hbm, xi_hbm, xj_hbm,
          idx_d, rows_d, idx_s, rows_s, sem_d, sem_s):
        wid = lax.axis_index("s") * NC + lax.axis_index("c")

        def body(t, carry):
            g = wid + NW * t

            @pl.when(g < n_groups)
            def _():
                base = g * G
                pltpu.sync_copy(dst_hbm.at[pl.ds(base, G)], idx_d)
                pltpu.sync_copy(src_hbm.at[pl.ds(base, G)], idx_s)
                cp_d = pltpu.async_copy(x_hbm.at[idx_d], rows_d, sem_d)
                cp_s = pltpu.async_copy(x_hbm.at[idx_s], rows_s, sem_s)
                cp_d.wait()
                cp_s.wait()
                pltpu.sync_copy(rows_d, xi_hbm.at[pl.ds(base, G)])
                pltpu.sync_copy(rows_s, xj_hbm.at[pl.ds(base, G)])

            return carry

        lax.fori_loop(0, n_iters, body, 0)

    return k(x, dst, src)


def _sc_scatter(msg, dst, zeros):
    E = msg.shape[0]
    n_groups = E // G
    n_iters = (n_groups + NW - 1) // NW
    rpt = NPAD // NS

    @functools.partial(
        pl.kernel,
        mesh=_sc_mesh(),
        out_type=jax.ShapeDtypeStruct((NC, NPAD, H), jnp.float32),
        scratch_types=[
            pltpu.VMEM((G,), jnp.int32),
            pltpu.VMEM((G, H), jnp.float32),
            pltpu.VMEM_SHARED((NPAD, H), jnp.float32),
            pltpu.SemaphoreType.DMA,
        ],
    )
    def k(msg_hbm, dst_hbm, zeros_hbm, out_hbm, idx_v, rows_v, acc_sh, sem):
        cid = lax.axis_index("c")
        sid = lax.axis_index("s")
        wid = sid * NC + cid

        pltpu.sync_copy(zeros_hbm.at[pl.ds(sid * rpt, rpt)],
                        acc_sh.at[pl.ds(sid * rpt, rpt)])
        plsc.subcore_barrier()

        def body(t, carry):
            g = wid + NW * t

            @pl.when(g < n_groups)
            def _():
                base = g * G
                pltpu.sync_copy(dst_hbm.at[pl.ds(base, G)], idx_v)
                pltpu.sync_copy(msg_hbm.at[pl.ds(base, G)], rows_v)
                pltpu.sync_copy(rows_v, acc_sh.at[idx_v], add=True)

            return carry

        lax.fori_loop(0, n_iters, body, 0)
        plsc.subcore_barrier()
        pltpu.sync_copy(acc_sh.at[pl.ds(sid * rpt, rpt)],
                        out_hbm.at[cid, pl.ds(sid * rpt, rpt)])

    return k(msg, dst, zeros)


def _tc_edge(xi, xj, ea, w0c, b0c, w1, b1, w2, b2, ln_g, ln_b, gw1, gb1):
    E = xi.shape[0]
    BE = 512
    grid = E // BE

    def body(xi_ref, xj_ref, ea_ref, w0c_ref, b0c_ref, w1_ref, b1_ref,
             w2_ref, b2_ref, lng_ref, lnb_ref, gw1_ref, gb1_ref,
             eout_ref, msg_ref):
        ea_blk = ea_ref[...]
        cat = jnp.concatenate([xi_ref[...], xj_ref[...], ea_blk], axis=1)
        h0 = jnp.dot(cat, w0c_ref[...], preferred_element_type=jnp.float32)
        h0 = jnp.maximum(h0 + b0c_ref[...], 0.0)
        h = h0[:, :H]
        gh = h0[:, H:]
        h = jnp.dot(h, w1_ref[...], preferred_element_type=jnp.float32)
        h = jnp.maximum(h + b1_ref[...], 0.0)
        h = jnp.dot(h, w2_ref[...], preferred_element_type=jnp.float32)
        h = h + b2_ref[...]
        m = jnp.mean(h, axis=1, keepdims=True)
        c = h - m
        v = jnp.mean(c * c, axis=1, keepdims=True)
        e_new = c * lax.rsqrt(v + 1e-5) * lng_ref[...] + lnb_ref[...]
        z = jnp.sum(gh * gw1_ref[...], axis=1, keepdims=True) + gb1_ref[0, 0]
        gate = 1.0 / (1.0 + jnp.exp(-z))
        eout_ref[...] = ea_blk + e_new
        msg_ref[...] = gate * e_new

    blk = lambda r: pl.BlockSpec((BE, H), lambda i: (i, 0))
    full = lambda shape: pl.BlockSpec(shape, lambda i: (0,) * len(shape))
    return pl.pallas_call(
        body,
        grid=(grid,),
        in_specs=[
            blk(0), blk(0), blk(0),
            full((3 * H, 2 * H)), full((1, 2 * H)),
            full((H, H)), full((1, H)),
            full((H, H)), full((1, H)),
            full((1, H)), full((1, H)),
            full((1, H)), full((1, 1)),
        ],
        out_specs=[blk(0), blk(0)],
        out_shape=[
            jax.ShapeDtypeStruct((E, H), jnp.float32),
            jax.ShapeDtypeStruct((E, H), jnp.float32),
        ],
    )(xi, xj, ea, w0c, b0c, w1, b1, w2, b2, ln_g, ln_b, gw1, gb1)


def _tc_node(x, a0, a1, w0, b0, w1, b1, w2, b2, ln_g, ln_b):
    N = x.shape[0]
    BN = 1000
    grid = N // BN

    def body(x_ref, a0_ref, a1_ref, w0_ref, b0_ref, w1_ref, b1_ref, w2_ref,
             b2_ref, lng_ref, lnb_ref, out_ref):
        x_blk = x_ref[...]
        aggr = a0_ref[...] + a1_ref[...]
        cat = jnp.concatenate([x_blk, aggr], axis=1)
        h = jnp.dot(cat, w0_ref[...], preferred_element_type=jnp.float32)
        h = jnp.maximum(h + b0_ref[...], 0.0)
        h = jnp.dot(h, w1_ref[...], preferred_element_type=jnp.float32)
        h = jnp.maximum(h + b1_ref[...], 0.0)
        h = jnp.dot(h, w2_ref[...], preferred_element_type=jnp.float32)
        h = h + b2_ref[...]
        m = jnp.mean(h, axis=1, keepdims=True)
        c = h - m
        v = jnp.mean(c * c, axis=1, keepdims=True)
        out_ref[...] = x_blk + (c * lax.rsqrt(v + 1e-5) * lng_ref[...]
                                + lnb_ref[...])

    blk = pl.BlockSpec((BN, H), lambda i: (i, 0))
    full = lambda shape: pl.BlockSpec(shape, lambda i: (0,) * len(shape))
    return pl.pallas_call(
        body,
        grid=(grid,),
        in_specs=[
            blk, blk, blk,
            full((2 * H, H)), full((1, H)),
            full((H, H)), full((1, H)),
            full((H, H)), full((1, H)),
            full((1, H)), full((1, H)),
        ],
        out_specs=blk,
        out_shape=jax.ShapeDtypeStruct((N, H), jnp.float32),
    )(x, a0, a1, w0, b0, w1, b1, w2, b2, ln_g, ln_b)


def kernel(x, edge_index, edge_attr, params):
    p = params
    src = edge_index[0]
    dst = edge_index[1]
    N = x.shape[0]

    xi, xj = _sc_gather(x, dst, src)

    r1 = lambda a: a.reshape(1, H)
    w0c = jnp.concatenate([p['e_w0'], p['g_w0']], axis=1)
    b0c = jnp.concatenate([p['e_b0'], p['g_b0']]).reshape(1, 2 * H)
    e_out, msg = _tc_edge(
        xi, xj, edge_attr,
        w0c, b0c, p['e_w1'], r1(p['e_b1']),
        p['e_w2'], r1(p['e_b2']), r1(p['e_ln_g']), r1(p['e_ln_b']),
        p['g_w1'].reshape(1, H), p['g_b1'].reshape(1, 1))

    zeros = jnp.zeros((NPAD, H), jnp.float32)
    parts = _sc_scatter(msg, dst, zeros)

    x_new = _tc_node(
        x, parts[0, :N], parts[1, :N],
        p['n_w0'], r1(p['n_b0']), p['n_w1'], r1(p['n_b1']),
        p['n_w2'], r1(p['n_b2']), r1(p['n_ln_g']), r1(p['n_ln_b']))

    return (x_new, e_out)


# speedup vs baseline: 1.4501x; 1.4501x over previous
import functools

import jax
import jax.numpy as jnp
from jax import lax
from jax.experimental import pallas as pl
from jax.experimental.pallas import tpu as pltpu
from jax.experimental.pallas import tpu_sc as plsc

H = 128
G = 128
NC = 2
NS = 16
NW = NC * NS
NPAD = 10240
NCHUNK = 4
BE = 640


def _sc_mesh():
    return plsc.VectorSubcoreMesh(core_axis_name="c", subcore_axis_name="s")


def _sc_gather(x, dst, src):
    E = dst.shape[0]
    n_groups = E // G
    n_iters = (n_groups + NW - 1) // NW

    @functools.partial(
        pl.kernel,
        mesh=_sc_mesh(),
        out_type=(
            jax.ShapeDtypeStruct((E, H), jnp.float32),
            jax.ShapeDtypeStruct((E, H), jnp.float32),
        ),
        scratch_types=[
            pltpu.VMEM((G,), jnp.int32),
            pltpu.VMEM((G, H), jnp.float32),
            pltpu.VMEM((G,), jnp.int32),
            pltpu.VMEM((G, H), jnp.float32),
            pltpu.SemaphoreType.DMA,
            pltpu.SemaphoreType.DMA,
        ],
    )
    def k(x_hbm, dst_hbm, src_hbm, xi_hbm, xj_hbm,
          idx_d, rows_d, idx_s, rows_s, sem_d, sem_s):
        wid = lax.axis_index("s") * NC + lax.axis_index("c")

        def body(t, carry):
            g = wid + NW * t

            @pl.when(g < n_groups)
            def _():
                base = g * G
                pltpu.sync_copy(dst_hbm.at[pl.ds(base, G)], idx_d)
                pltpu.sync_copy(src_hbm.at[pl.ds(base, G)], idx_s)
                cp_d = pltpu.async_copy(x_hbm.at[idx_d], rows_d, sem_d)
                cp_s = pltpu.async_copy(x_hbm.at[idx_s], rows_s, sem_s)
                cp_d.wait()
                cp_s.wait()
                pltpu.sync_copy(rows_d, xi_hbm.at[pl.ds(base, G)])
                pltpu.sync_copy(rows_s, xj_hbm.at[pl.ds(base, G)])

            return carry

        lax.fori_loop(0, n_iters, body, 0)

    return k(x, dst, src)


def _sc_scatter(msg, dst, prev):
    E = msg.shape[0]
    n_groups = E // G
    n_iters = (n_groups + NW - 1) // NW
    rpt = NPAD // NS

    @functools.partial(
        pl.kernel,
        mesh=_sc_mesh(),
        out_type=jax.ShapeDtypeStruct((NC, NPAD, H), jnp.float32),
        scratch_types=[
            pltpu.VMEM((G,), jnp.int32),
            pltpu.VMEM((G, H), jnp.float32),
            pltpu.VMEM_SHARED((NPAD, H), jnp.float32),
            pltpu.SemaphoreType.DMA,
        ],
    )
    def k(msg_hbm, dst_hbm, prev_hbm, out_hbm, idx_v, rows_v, acc_sh, sem):
        cid = lax.axis_index("c")
        sid = lax.axis_index("s")
        wid = sid * NC + cid

        pltpu.sync_copy(prev_hbm.at[cid, pl.ds(sid * rpt, rpt)],
                        acc_sh.at[pl.ds(sid * rpt, rpt)])
        plsc.subcore_barrier()

        def body(t, carry):
            g = wid + NW * t

            @pl.when(g < n_groups)
            def _():
                base = g * G
                pltpu.sync_copy(dst_hbm.at[pl.ds(base, G)], idx_v)
                pltpu.sync_copy(msg_hbm.at[pl.ds(base, G)], rows_v)
                pltpu.sync_copy(rows_v, acc_sh.at[idx_v], add=True)

            return carry

        lax.fori_loop(0, n_iters, body, 0)
        plsc.subcore_barrier()
        pltpu.sync_copy(acc_sh.at[pl.ds(sid * rpt, rpt)],
                        out_hbm.at[cid, pl.ds(sid * rpt, rpt)])

    return k(msg, dst, prev)


def _tc_edge_chunk(xi, xj, ea_full, w0c, b0c, w1, b1, w2, b2, ln_g, ln_b,
                   gw1, gb1, eout_prev, blk_off):
    Ec = xi.shape[0]
    grid = Ec // BE

    def body(xi_ref, xj_ref, ea_ref, w0c_ref, b0c_ref, w1_ref, b1_ref,
             w2_ref, b2_ref, lng_ref, lnb_ref, gw1_ref, gb1_ref,
             *rest_refs):
        eout_ref, msg_ref = rest_refs[-2], rest_refs[-1]
        ea_blk = ea_ref[...]
        cat = jnp.concatenate([xi_ref[...], xj_ref[...], ea_blk], axis=1)
        h0 = jnp.dot(cat, w0c_ref[...], preferred_element_type=jnp.float32)
        h0 = jnp.maximum(h0 + b0c_ref[...], 0.0)
        h = h0[:, :H]
        gh = h0[:, H:]
        h = jnp.dot(h, w1_ref[...], preferred_element_type=jnp.float32)
        h = jnp.maximum(h + b1_ref[...], 0.0)
        h = jnp.dot(h, w2_ref[...], preferred_element_type=jnp.float32)
        h = h + b2_ref[...]
        m = jnp.mean(h, axis=1, keepdims=True)
        c = h - m
        v = jnp.mean(c * c, axis=1, keepdims=True)
        e_new = c * lax.rsqrt(v + 1e-5) * lng_ref[...] + lnb_ref[...]
        z = jnp.sum(gh * gw1_ref[...], axis=1, keepdims=True) + gb1_ref[0, 0]
        gate = 1.0 / (1.0 + jnp.exp(-z))
        eout_ref[...] = ea_blk + e_new
        msg_ref[...] = gate * e_new

    E = ea_full.shape[0]
    blk = pl.BlockSpec((BE, H), lambda i: (i, 0))
    off = pl.BlockSpec((BE, H), lambda i: (i + blk_off, 0))
    full = lambda shape: pl.BlockSpec(shape, lambda i: (0,) * len(shape))
    in_specs = [
        blk, blk, off,
        full((3 * H, 2 * H)), full((1, 2 * H)),
        full((H, H)), full((1, H)),
        full((H, H)), full((1, H)),
        full((1, H)), full((1, H)),
        full((1, H)), full((1, 1)),
    ]
    args = [xi, xj, ea_full, w0c, b0c, w1, b1, w2, b2, ln_g, ln_b, gw1, gb1]
    aliases = {}
    if eout_prev is not None:
        in_specs.append(pl.BlockSpec(memory_space=pl.ANY))
        args.append(eout_prev)
        aliases = {13: 0}

        def body_alias(*refs):
            body(*refs[:13], *refs[14:])

        body_fn = body_alias
    else:
        body_fn = body
    return pl.pallas_call(
        body_fn,
        grid=(grid,),
        in_specs=in_specs,
        out_specs=[off, blk],
        out_shape=[
            jax.ShapeDtypeStruct((E, H), jnp.float32),
            jax.ShapeDtypeStruct((Ec, H), jnp.float32),
        ],
        input_output_aliases=aliases,
    )(*args)


def _tc_node(x, a0, a1, w0, b0, w1, b1, w2, b2, ln_g, ln_b):
    N = x.shape[0]
    BN = 1000
    grid = N // BN

    def body(x_ref, a0_ref, a1_ref, w0_ref, b0_ref, w1_ref, b1_ref, w2_ref,
             b2_ref, lng_ref, lnb_ref, out_ref):
        x_blk = x_ref[...]
        aggr = a0_ref[...] + a1_ref[...]
        cat = jnp.concatenate([x_blk, aggr], axis=1)
        h = jnp.dot(cat, w0_ref[...], preferred_element_type=jnp.float32)
        h = jnp.maximum(h + b0_ref[...], 0.0)
        h = jnp.dot(h, w1_ref[...], preferred_element_type=jnp.float32)
        h = jnp.maximum(h + b1_ref[...], 0.0)
        h = jnp.dot(h, w2_ref[...], preferred_element_type=jnp.float32)
        h = h + b2_ref[...]
        m = jnp.mean(h, axis=1, keepdims=True)
        c = h - m
        v = jnp.mean(c * c, axis=1, keepdims=True)
        out_ref[...] = x_blk + (c * lax.rsqrt(v + 1e-5) * lng_ref[...]
                                + lnb_ref[...])

    blk = pl.BlockSpec((BN, H), lambda i: (i, 0))
    full = lambda shape: pl.BlockSpec(shape, lambda i: (0,) * len(shape))
    return pl.pallas_call(
        body,
        grid=(grid,),
        in_specs=[
            blk, blk, blk,
            full((2 * H, H)), full((1, H)),
            full((H, H)), full((1, H)),
            full((H, H)), full((1, H)),
            full((1, H)), full((1, H)),
        ],
        out_specs=blk,
        out_shape=jax.ShapeDtypeStruct((N, H), jnp.float32),
    )(x, a0, a1, w0, b0, w1, b1, w2, b2, ln_g, ln_b)


def kernel(x, edge_index, edge_attr, params):
    p = params
    src = edge_index[0]
    dst = edge_index[1]
    N = x.shape[0]
    E = edge_attr.shape[0]
    Ec = E // NCHUNK

    r1 = lambda a: a.reshape(1, H)
    w0c = jnp.concatenate([p['e_w0'], p['g_w0']], axis=1)
    b0c = jnp.concatenate([p['e_b0'], p['g_b0']]).reshape(1, 2 * H)

    gathered = []
    for k in range(NCHUNK):
        sl = slice(k * Ec, (k + 1) * Ec)
        gathered.append((_sc_gather(x, dst[sl], src[sl]), dst[sl]))

    eout = None
    msgs = []
    for k, ((xi, xj), _) in enumerate(gathered):
        eout, msg = _tc_edge_chunk(
            xi, xj, edge_attr, w0c, b0c, p['e_w1'], r1(p['e_b1']),
            p['e_w2'], r1(p['e_b2']), r1(p['e_ln_g']), r1(p['e_ln_b']),
            p['g_w1'].reshape(1, H), p['g_b1'].reshape(1, 1),
            eout, k * (Ec // BE))
        msgs.append(msg)

    parts = jnp.zeros((NC, NPAD, H), jnp.float32)
    for k, (_, dst_k) in enumerate(gathered):
        parts = _sc_scatter(msgs[k], dst_k, parts)

    x_new = _tc_node(
        x, parts[0, :N], parts[1, :N],
        p['n_w0'], r1(p['n_b0']), p['n_w1'], r1(p['n_b1']),
        p['n_w2'], r1(p['n_b2']), r1(p['n_ln_g']), r1(p['n_ln_b']))

    return (x_new, eout)


# speedup vs baseline: 1.4898x; 1.0273x over previous
import functools

import jax
import jax.numpy as jnp
from jax import lax
from jax.experimental import pallas as pl
from jax.experimental.pallas import tpu as pltpu
from jax.experimental.pallas import tpu_sc as plsc

H = 128
G = 128
NC = 2
NS = 16
NW = NC * NS
NPAD = 10240
NCHUNK = 4
BE = 640


def _sc_mesh():
    return plsc.VectorSubcoreMesh(core_axis_name="c", subcore_axis_name="s")


def _sc_gather(x, dst, src):
    E = dst.shape[0]
    n_groups = E // G
    n_iters = (n_groups + NW - 1) // NW

    @functools.partial(
        pl.kernel,
        mesh=_sc_mesh(),
        out_type=(
            jax.ShapeDtypeStruct((E, H), jnp.float32),
            jax.ShapeDtypeStruct((E, H), jnp.float32),
        ),
        scratch_types=[
            pltpu.VMEM((2, G), jnp.int32),
            pltpu.VMEM((2, G, H), jnp.float32),
            pltpu.VMEM((2, G), jnp.int32),
            pltpu.VMEM((2, G, H), jnp.float32),
            pltpu.SemaphoreType.DMA((2,)),
            pltpu.SemaphoreType.DMA((2,)),
        ],
    )
    def k(x_hbm, dst_hbm, src_hbm, xi_hbm, xj_hbm,
          idx_d, rows_d, idx_s, rows_s, sem_d, sem_s):
        wid = lax.axis_index("s") * NC + lax.axis_index("c")
        nv = (n_groups - wid + NW - 1) // NW

        def prefetch(t, slot):
            base = (wid + NW * t) * G
            pltpu.sync_copy(dst_hbm.at[pl.ds(base, G)], idx_d.at[slot])
            pltpu.sync_copy(src_hbm.at[pl.ds(base, G)], idx_s.at[slot])
            pltpu.async_copy(x_hbm.at[idx_d.at[slot]], rows_d.at[slot],
                             sem_d.at[slot])
            pltpu.async_copy(x_hbm.at[idx_s.at[slot]], rows_s.at[slot],
                             sem_s.at[slot])

        @pl.when(nv > 0)
        def _():
            prefetch(0, 0)

        def body(t, carry):
            p = lax.rem(t, 2)

            @pl.when(t + 1 < nv)
            def _():
                prefetch(t + 1, 1 - p)

            @pl.when(t < nv)
            def _():
                base = (wid + NW * t) * G
                pltpu.make_async_copy(x_hbm.at[idx_d.at[p]], rows_d.at[p],
                                      sem_d.at[p]).wait()
                pltpu.make_async_copy(x_hbm.at[idx_s.at[p]], rows_s.at[p],
                                      sem_s.at[p]).wait()
                pltpu.sync_copy(rows_d.at[p], xi_hbm.at[pl.ds(base, G)])
                pltpu.sync_copy(rows_s.at[p], xj_hbm.at[pl.ds(base, G)])

            return carry

        lax.fori_loop(0, n_iters, body, 0)

    return k(x, dst, src)


def _sc_scatter(msg, dst, prev):
    E = msg.shape[0]
    n_groups = E // G
    n_iters = (n_groups + NW - 1) // NW
    rpt = NPAD // NS

    @functools.partial(
        pl.kernel,
        mesh=_sc_mesh(),
        out_type=jax.ShapeDtypeStruct((NC, NPAD, H), jnp.float32),
        scratch_types=[
            pltpu.VMEM((2, G), jnp.int32),
            pltpu.VMEM((2, G, H), jnp.float32),
            pltpu.VMEM_SHARED((NPAD, H), jnp.float32),
            pltpu.SemaphoreType.DMA((2,)),
        ],
    )
    def k(msg_hbm, dst_hbm, prev_hbm, out_hbm, idx_v, rows_v, acc_sh, sem):
        cid = lax.axis_index("c")
        sid = lax.axis_index("s")
        wid = sid * NC + cid
        nv = (n_groups - wid + NW - 1) // NW

        def prefetch(t, slot):
            base = (wid + NW * t) * G
            pltpu.sync_copy(dst_hbm.at[pl.ds(base, G)], idx_v.at[slot])
            pltpu.async_copy(msg_hbm.at[pl.ds(base, G)], rows_v.at[slot],
                             sem.at[slot])

        pltpu.sync_copy(prev_hbm.at[cid, pl.ds(sid * rpt, rpt)],
                        acc_sh.at[pl.ds(sid * rpt, rpt)])
        plsc.subcore_barrier()

        @pl.when(nv > 0)
        def _():
            prefetch(0, 0)

        def body(t, carry):
            p = lax.rem(t, 2)

            @pl.when(t + 1 < nv)
            def _():
                prefetch(t + 1, 1 - p)

            @pl.when(t < nv)
            def _():
                base = (wid + NW * t) * G
                pltpu.make_async_copy(msg_hbm.at[pl.ds(base, G)],
                                      rows_v.at[p], sem.at[p]).wait()
                pltpu.sync_copy(rows_v.at[p], acc_sh.at[idx_v.at[p]],
                                add=True)

            return carry

        lax.fori_loop(0, n_iters, body, 0)
        plsc.subcore_barrier()
        pltpu.sync_copy(acc_sh.at[pl.ds(sid * rpt, rpt)],
                        out_hbm.at[cid, pl.ds(sid * rpt, rpt)])

    return k(msg, dst, prev)


def _tc_edge_chunk(xi, xj, ea_full, w0c, b0c, w1, b1, w2, b2, ln_g, ln_b,
                   gw1, gb1, eout_prev, blk_off):
    Ec = xi.shape[0]
    grid = Ec // BE

    def body(xi_ref, xj_ref, ea_ref, w0c_ref, b0c_ref, w1_ref, b1_ref,
             w2_ref, b2_ref, lng_ref, lnb_ref, gw1_ref, gb1_ref,
             *rest_refs):
        eout_ref, msg_ref = rest_refs[-2], rest_refs[-1]
        ea_blk = ea_ref[...]
        cat = jnp.concatenate([xi_ref[...], xj_ref[...], ea_blk], axis=1)
        h0 = jnp.dot(cat, w0c_ref[...], preferred_element_type=jnp.float32)
        h0 = jnp.maximum(h0 + b0c_ref[...], 0.0)
        h = h0[:, :H]
        gh = h0[:, H:]
        h = jnp.dot(h, w1_ref[...], preferred_element_type=jnp.float32)
        h = jnp.maximum(h + b1_ref[...], 0.0)
        h = jnp.dot(h, w2_ref[...], preferred_element_type=jnp.float32)
        h = h + b2_ref[...]
        m = jnp.mean(h, axis=1, keepdims=True)
        c = h - m
        v = jnp.mean(c * c, axis=1, keepdims=True)
        e_new = c * lax.rsqrt(v + 1e-5) * lng_ref[...] + lnb_ref[...]
        z = jnp.sum(gh * gw1_ref[...], axis=1, keepdims=True) + gb1_ref[0, 0]
        gate = 1.0 / (1.0 + jnp.exp(-z))
        eout_ref[...] = ea_blk + e_new
        msg_ref[...] = gate * e_new

    E = ea_full.shape[0]
    blk = pl.BlockSpec((BE, H), lambda i: (i, 0))
    off = pl.BlockSpec((BE, H), lambda i: (i + blk_off, 0))
    full = lambda shape: pl.BlockSpec(shape, lambda i: (0,) * len(shape))
    in_specs = [
        blk, blk, off,
        full((3 * H, 2 * H)), full((1, 2 * H)),
        full((H, H)), full((1, H)),
        full((H, H)), full((1, H)),
        full((1, H)), full((1, H)),
        full((1, H)), full((1, 1)),
    ]
    args = [xi, xj, ea_full, w0c, b0c, w1, b1, w2, b2, ln_g, ln_b, gw1, gb1]
    aliases = {}
    if eout_prev is not None:
        in_specs.append(pl.BlockSpec(memory_space=pl.ANY))
        args.append(eout_prev)
        aliases = {13: 0}

        def body_alias(*refs):
            body(*refs[:13], *refs[14:])

        body_fn = body_alias
    else:
        body_fn = body
    return pl.pallas_call(
        body_fn,
        grid=(grid,),
        in_specs=in_specs,
        out_specs=[off, blk],
        out_shape=[
            jax.ShapeDtypeStruct((E, H), jnp.float32),
            jax.ShapeDtypeStruct((Ec, H), jnp.float32),
        ],
        input_output_aliases=aliases,
    )(*args)


def _tc_node(x, a0, a1, w0, b0, w1, b1, w2, b2, ln_g, ln_b):
    N = x.shape[0]
    BN = 1000
    grid = N // BN

    def body(x_ref, a0_ref, a1_ref, w0_ref, b0_ref, w1_ref, b1_ref, w2_ref,
             b2_ref, lng_ref, lnb_ref, out_ref):
        x_blk = x_ref[...]
        aggr = a0_ref[...] + a1_ref[...]
        cat = jnp.concatenate([x_blk, aggr], axis=1)
        h = jnp.dot(cat, w0_ref[...], preferred_element_type=jnp.float32)
        h = jnp.maximum(h + b0_ref[...], 0.0)
        h = jnp.dot(h, w1_ref[...], preferred_element_type=jnp.float32)
        h = jnp.maximum(h + b1_ref[...], 0.0)
        h = jnp.dot(h, w2_ref[...], preferred_element_type=jnp.float32)
        h = h + b2_ref[...]
        m = jnp.mean(h, axis=1, keepdims=True)
        c = h - m
        v = jnp.mean(c * c, axis=1, keepdims=True)
        out_ref[...] = x_blk + (c * lax.rsqrt(v + 1e-5) * lng_ref[...]
                                + lnb_ref[...])

    blk = pl.BlockSpec((BN, H), lambda i: (i, 0))
    full = lambda shape: pl.BlockSpec(shape, lambda i: (0,) * len(shape))
    return pl.pallas_call(
        body,
        grid=(grid,),
        in_specs=[
            blk, blk, blk,
            full((2 * H, H)), full((1, H)),
            full((H, H)), full((1, H)),
            full((H, H)), full((1, H)),
            full((1, H)), full((1, H)),
        ],
        out_specs=blk,
        out_shape=jax.ShapeDtypeStruct((N, H), jnp.float32),
    )(x, a0, a1, w0, b0, w1, b1, w2, b2, ln_g, ln_b)


def kernel(x, edge_index, edge_attr, params):
    p = params
    src = edge_index[0]
    dst = edge_index[1]
    N = x.shape[0]
    E = edge_attr.shape[0]
    Ec = E // NCHUNK

    r1 = lambda a: a.reshape(1, H)
    w0c = jnp.concatenate([p['e_w0'], p['g_w0']], axis=1)
    b0c = jnp.concatenate([p['e_b0'], p['g_b0']]).reshape(1, 2 * H)

    gathered = []
    for k in range(NCHUNK):
        sl = slice(k * Ec, (k + 1) * Ec)
        gathered.append((_sc_gather(x, dst[sl], src[sl]), dst[sl]))

    eout = None
    msgs = []
    for k, ((xi, xj), _) in enumerate(gathered):
        eout, msg = _tc_edge_chunk(
            xi, xj, edge_attr, w0c, b0c, p['e_w1'], r1(p['e_b1']),
            p['e_w2'], r1(p['e_b2']), r1(p['e_ln_g']), r1(p['e_ln_b']),
            p['g_w1'].reshape(1, H), p['g_b1'].reshape(1, 1),
            eout, k * (Ec // BE))
        msgs.append(msg)

    parts = jnp.zeros((NC, NPAD, H), jnp.float32)
    for k, (_, dst_k) in enumerate(gathered):
        parts = _sc_scatter(msgs[k], dst_k, parts)

    x_new = _tc_node(
        x, parts[0, :N], parts[1, :N],
        p['n_w0'], r1(p['n_b0']), p['n_w1'], r1(p['n_b1']),
        p['n_w2'], r1(p['n_b2']), r1(p['n_ln_g']), r1(p['n_ln_b']))

    return (x_new, eout)


# speedup vs baseline: 1.8512x; 1.2426x over previous
import functools

import jax
import jax.numpy as jnp
from jax import lax
from jax.experimental import pallas as pl
from jax.experimental.pallas import tpu as pltpu
from jax.experimental.pallas import tpu_sc as plsc

H = 128
G = 128
NC = 2
NS = 16
NW = NC * NS
NPAD = 10240
NCHUNK = 5
BE = 1280


def _sc_mesh():
    return plsc.VectorSubcoreMesh(core_axis_name="c", subcore_axis_name="s")


def _sc_gather(x, dst, src):
    E = dst.shape[0]
    n_groups = E // G
    n_iters = (n_groups + NW - 1) // NW

    @functools.partial(
        pl.kernel,
        mesh=_sc_mesh(),
        out_type=(
            jax.ShapeDtypeStruct((E, H), jnp.float32),
            jax.ShapeDtypeStruct((E, H), jnp.float32),
        ),
        scratch_types=[
            pltpu.VMEM((2, G), jnp.int32),
            pltpu.VMEM((2, G, H), jnp.float32),
            pltpu.VMEM((2, G), jnp.int32),
            pltpu.VMEM((2, G, H), jnp.float32),
            pltpu.SemaphoreType.DMA((2,)),
            pltpu.SemaphoreType.DMA((2,)),
        ],
    )
    def k(x_hbm, dst_hbm, src_hbm, xi_hbm, xj_hbm,
          idx_d, rows_d, idx_s, rows_s, sem_d, sem_s):
        wid = lax.axis_index("s") * NC + lax.axis_index("c")
        nv = (n_groups - wid + NW - 1) // NW

        def prefetch(t, slot):
            base = (wid + NW * t) * G
            pltpu.sync_copy(dst_hbm.at[pl.ds(base, G)], idx_d.at[slot])
            pltpu.sync_copy(src_hbm.at[pl.ds(base, G)], idx_s.at[slot])
            pltpu.async_copy(x_hbm.at[idx_d.at[slot]], rows_d.at[slot],
                             sem_d.at[slot])
            pltpu.async_copy(x_hbm.at[idx_s.at[slot]], rows_s.at[slot],
                             sem_s.at[slot])

        @pl.when(nv > 0)
        def _():
            prefetch(0, 0)

        def body(t, carry):
            p = lax.rem(t, 2)

            @pl.when(t + 1 < nv)
            def _():
                prefetch(t + 1, 1 - p)

            @pl.when(t < nv)
            def _():
                base = (wid + NW * t) * G
                pltpu.make_async_copy(x_hbm.at[idx_d.at[p]], rows_d.at[p],
                                      sem_d.at[p]).wait()
                pltpu.make_async_copy(x_hbm.at[idx_s.at[p]], rows_s.at[p],
                                      sem_s.at[p]).wait()
                pltpu.sync_copy(rows_d.at[p], xi_hbm.at[pl.ds(base, G)])
                pltpu.sync_copy(rows_s.at[p], xj_hbm.at[pl.ds(base, G)])

            return carry

        lax.fori_loop(0, n_iters, body, 0)

    return k(x, dst, src)


def _sc_scatter(msg, dst, prev):
    E = msg.shape[0]
    n_groups = E // G
    n_iters = (n_groups + NW - 1) // NW
    rpt = NPAD // NS

    @functools.partial(
        pl.kernel,
        mesh=_sc_mesh(),
        out_type=jax.ShapeDtypeStruct((NC, NPAD, H), jnp.float32),
        scratch_types=[
            pltpu.VMEM((2, G), jnp.int32),
            pltpu.VMEM((2, G, H), jnp.float32),
            pltpu.VMEM_SHARED((NPAD, H), jnp.float32),
            pltpu.SemaphoreType.DMA((2,)),
        ],
    )
    def k(msg_hbm, dst_hbm, prev_hbm, out_hbm, idx_v, rows_v, acc_sh, sem):
        cid = lax.axis_index("c")
        sid = lax.axis_index("s")
        wid = sid * NC + cid
        nv = (n_groups - wid + NW - 1) // NW

        def prefetch(t, slot):
            base = (wid + NW * t) * G
            pltpu.sync_copy(dst_hbm.at[pl.ds(base, G)], idx_v.at[slot])
            pltpu.async_copy(msg_hbm.at[pl.ds(base, G)], rows_v.at[slot],
                             sem.at[slot])

        pltpu.sync_copy(prev_hbm.at[cid, pl.ds(sid * rpt, rpt)],
                        acc_sh.at[pl.ds(sid * rpt, rpt)])
        plsc.subcore_barrier()

        @pl.when(nv > 0)
        def _():
            prefetch(0, 0)

        def body(t, carry):
            p = lax.rem(t, 2)

            @pl.when(t + 1 < nv)
            def _():
                prefetch(t + 1, 1 - p)

            @pl.when(t < nv)
            def _():
                base = (wid + NW * t) * G
                pltpu.make_async_copy(msg_hbm.at[pl.ds(base, G)],
                                      rows_v.at[p], sem.at[p]).wait()
                pltpu.sync_copy(rows_v.at[p], acc_sh.at[idx_v.at[p]],
                                add=True)

            return carry

        lax.fori_loop(0, n_iters, body, 0)
        plsc.subcore_barrier()
        pltpu.sync_copy(acc_sh.at[pl.ds(sid * rpt, rpt)],
                        out_hbm.at[cid, pl.ds(sid * rpt, rpt)])

    return k(msg, dst, prev)


def _tc_edge_chunk(xi, xj, ea_full, w0c, b0c, w1, b1, w2, b2, ln_g, ln_b,
                   gw1, gb1, eout_prev, blk_off):
    Ec = xi.shape[0]
    grid = Ec // BE

    def body(xi_ref, xj_ref, ea_ref, w0c_ref, b0c_ref, w1_ref, b1_ref,
             w2_ref, b2_ref, lng_ref, lnb_ref, gw1_ref, gb1_ref,
             *rest_refs):
        eout_ref, msg_ref = rest_refs[-2], rest_refs[-1]
        ea_blk = ea_ref[...]
        cat = jnp.concatenate([xi_ref[...], xj_ref[...], ea_blk], axis=1)
        h0 = jnp.dot(cat, w0c_ref[...], preferred_element_type=jnp.float32)
        h0 = jnp.maximum(h0 + b0c_ref[...], 0.0)
        h = h0[:, :H]
        gh = h0[:, H:]
        h = jnp.dot(h, w1_ref[...], preferred_element_type=jnp.float32)
        h = jnp.maximum(h + b1_ref[...], 0.0)
        h = jnp.dot(h, w2_ref[...], preferred_element_type=jnp.float32)
        h = h + b2_ref[...]
        m = jnp.mean(h, axis=1, keepdims=True)
        c = h - m
        v = jnp.mean(c * c, axis=1, keepdims=True)
        e_new = c * lax.rsqrt(v + 1e-5) * lng_ref[...] + lnb_ref[...]
        z = jnp.sum(gh * gw1_ref[...], axis=1, keepdims=True) + gb1_ref[0, 0]
        gate = 1.0 / (1.0 + jnp.exp(-z))
        eout_ref[...] = ea_blk + e_new
        msg_ref[...] = gate * e_new

    E = ea_full.shape[0]
    blk = pl.BlockSpec((BE, H), lambda i: (i, 0))
    off = pl.BlockSpec((BE, H), lambda i: (i + blk_off, 0))
    full = lambda shape: pl.BlockSpec(shape, lambda i: (0,) * len(shape))
    in_specs = [
        blk, blk, off,
        full((3 * H, 2 * H)), full((1, 2 * H)),
        full((H, H)), full((1, H)),
        full((H, H)), full((1, H)),
        full((1, H)), full((1, H)),
        full((1, H)), full((1, 1)),
    ]
    args = [xi, xj, ea_full, w0c, b0c, w1, b1, w2, b2, ln_g, ln_b, gw1, gb1]
    aliases = {}
    if eout_prev is not None:
        in_specs.append(pl.BlockSpec(memory_space=pl.ANY))
        args.append(eout_prev)
        aliases = {13: 0}

        def body_alias(*refs):
            body(*refs[:13], *refs[14:])

        body_fn = body_alias
    else:
        body_fn = body
    return pl.pallas_call(
        body_fn,
        grid=(grid,),
        in_specs=in_specs,
        out_specs=[off, blk],
        out_shape=[
            jax.ShapeDtypeStruct((E, H), jnp.float32),
            jax.ShapeDtypeStruct((Ec, H), jnp.float32),
        ],
        input_output_aliases=aliases,
    )(*args)


def _tc_node(x, a0, a1, w0, b0, w1, b1, w2, b2, ln_g, ln_b):
    N = x.shape[0]
    BN = 1000
    grid = N // BN

    def body(x_ref, a0_ref, a1_ref, w0_ref, b0_ref, w1_ref, b1_ref, w2_ref,
             b2_ref, lng_ref, lnb_ref, out_ref):
        x_blk = x_ref[...]
        aggr = a0_ref[...] + a1_ref[...]
        cat = jnp.concatenate([x_blk, aggr], axis=1)
        h = jnp.dot(cat, w0_ref[...], preferred_element_type=jnp.float32)
        h = jnp.maximum(h + b0_ref[...], 0.0)
        h = jnp.dot(h, w1_ref[...], preferred_element_type=jnp.float32)
        h = jnp.maximum(h + b1_ref[...], 0.0)
        h = jnp.dot(h, w2_ref[...], preferred_element_type=jnp.float32)
        h = h + b2_ref[...]
        m = jnp.mean(h, axis=1, keepdims=True)
        c = h - m
        v = jnp.mean(c * c, axis=1, keepdims=True)
        out_ref[...] = x_blk + (c * lax.rsqrt(v + 1e-5) * lng_ref[...]
                                + lnb_ref[...])

    blk = pl.BlockSpec((BN, H), lambda i: (i, 0))
    full = lambda shape: pl.BlockSpec(shape, lambda i: (0,) * len(shape))
    return pl.pallas_call(
        body,
        grid=(grid,),
        in_specs=[
            blk, blk, blk,
            full((2 * H, H)), full((1, H)),
            full((H, H)), full((1, H)),
            full((H, H)), full((1, H)),
            full((1, H)), full((1, H)),
        ],
        out_specs=blk,
        out_shape=jax.ShapeDtypeStruct((N, H), jnp.float32),
    )(x, a0, a1, w0, b0, w1, b1, w2, b2, ln_g, ln_b)


def kernel(x, edge_index, edge_attr, params):
    p = params
    src = edge_index[0]
    dst = edge_index[1]
    N = x.shape[0]
    E = edge_attr.shape[0]
    Ec = E // NCHUNK

    r1 = lambda a: a.reshape(1, H)
    w0c = jnp.concatenate([p['e_w0'], p['g_w0']], axis=1)
    b0c = jnp.concatenate([p['e_b0'], p['g_b0']]).reshape(1, 2 * H)

    gathered = []
    for k in range(NCHUNK):
        sl = slice(k * Ec, (k + 1) * Ec)
        gathered.append((_sc_gather(x, dst[sl], src[sl]), dst[sl]))

    eout = None
    msgs = []
    for k, ((xi, xj), _) in enumerate(gathered):
        eout, msg = _tc_edge_chunk(
            xi, xj, edge_attr, w0c, b0c, p['e_w1'], r1(p['e_b1']),
            p['e_w2'], r1(p['e_b2']), r1(p['e_ln_g']), r1(p['e_ln_b']),
            p['g_w1'].reshape(1, H), p['g_b1'].reshape(1, 1),
            eout, k * (Ec // BE))
        msgs.append(msg)

    parts = jnp.zeros((NC, NPAD, H), jnp.float32)
    for k, (_, dst_k) in enumerate(gathered):
        parts = _sc_scatter(msgs[k], dst_k, parts)

    x_new = _tc_node(
        x, parts[0, :N], parts[1, :N],
        p['n_w0'], r1(p['n_b0']), p['n_w1'], r1(p['n_b1']),
        p['n_w2'], r1(p['n_b2']), r1(p['n_ln_g']), r1(p['n_ln_b']))

    return (x_new, eout)


# speedup vs baseline: 1.8700x; 1.0101x over previous
import functools

import jax
import jax.numpy as jnp
from jax import lax
from jax.experimental import pallas as pl
from jax.experimental.pallas import tpu as pltpu
from jax.experimental.pallas import tpu_sc as plsc

H = 128
G = 128
NC = 2
NS = 16
NW = NC * NS
NPAD = 10240
NCHUNK = 5
BE = 1280


def _sc_mesh():
    return plsc.VectorSubcoreMesh(core_axis_name="c", subcore_axis_name="s")


def _sc_gather(x_pad, dst, src):
    GG = G // 2
    E = dst.shape[0]
    n_groups = E // GG
    n_iters = (n_groups + NW - 1) // NW
    rpt = NPAD // NS

    @functools.partial(
        pl.kernel,
        mesh=_sc_mesh(),
        out_type=(
            jax.ShapeDtypeStruct((E, H), jnp.float32),
            jax.ShapeDtypeStruct((E, H), jnp.float32),
        ),
        scratch_types=[
            pltpu.VMEM((2, GG), jnp.int32),
            pltpu.VMEM((2, GG, H), jnp.float32),
            pltpu.VMEM((2, GG), jnp.int32),
            pltpu.VMEM((2, GG, H), jnp.float32),
            pltpu.VMEM_SHARED((NPAD, H), jnp.float32),
            pltpu.SemaphoreType.DMA((2,)),
            pltpu.SemaphoreType.DMA((2,)),
        ],
    )
    def k(x_hbm, dst_hbm, src_hbm, xi_hbm, xj_hbm,
          idx_d, rows_d, idx_s, rows_s, x_sh, sem_d, sem_s):
        sid = lax.axis_index("s")
        wid = sid * NC + lax.axis_index("c")
        nv = (n_groups - wid + NW - 1) // NW

        pltpu.sync_copy(x_hbm.at[pl.ds(sid * rpt, rpt)],
                        x_sh.at[pl.ds(sid * rpt, rpt)])
        plsc.subcore_barrier()

        def wait_store(t, p):
            pb = (wid + NW * t) * GG
            pltpu.make_async_copy(rows_d.at[p], xi_hbm.at[pl.ds(pb, GG)],
                                  sem_d.at[p]).wait()
            pltpu.make_async_copy(rows_s.at[p], xj_hbm.at[pl.ds(pb, GG)],
                                  sem_s.at[p]).wait()

        def body(t, carry):
            p = lax.rem(t, 2)

            @pl.when(t < nv)
            def _():
                base = (wid + NW * t) * GG
                pltpu.sync_copy(dst_hbm.at[pl.ds(base, GG)], idx_d.at[p])
                pltpu.sync_copy(src_hbm.at[pl.ds(base, GG)], idx_s.at[p])

                @pl.when(t >= 2)
                def _():
                    wait_store(t - 2, p)

                pltpu.sync_copy(x_sh.at[idx_d.at[p]], rows_d.at[p])
                pltpu.sync_copy(x_sh.at[idx_s.at[p]], rows_s.at[p])
                pltpu.async_copy(rows_d.at[p], xi_hbm.at[pl.ds(base, GG)],
                                 sem_d.at[p])
                pltpu.async_copy(rows_s.at[p], xj_hbm.at[pl.ds(base, GG)],
                                 sem_s.at[p])

            return carry

        lax.fori_loop(0, n_iters, body, 0)

        @pl.when(nv >= 2)
        def _():
            wait_store(nv - 2, lax.rem(nv - 2, 2))

        @pl.when(nv >= 1)
        def _():
            wait_store(nv - 1, lax.rem(nv - 1, 2))

    return k(x_pad, dst, src)


def _sc_scatter(msg, dst, prev):
    E = msg.shape[0]
    n_groups = E // G
    n_iters = (n_groups + NW - 1) // NW
    rpt = NPAD // NS

    @functools.partial(
        pl.kernel,
        mesh=_sc_mesh(),
        out_type=jax.ShapeDtypeStruct((NC, NPAD, H), jnp.float32),
        scratch_types=[
            pltpu.VMEM((2, G), jnp.int32),
            pltpu.VMEM((2, G, H), jnp.float32),
            pltpu.VMEM_SHARED((NPAD, H), jnp.float32),
            pltpu.SemaphoreType.DMA((2,)),
        ],
    )
    def k(msg_hbm, dst_hbm, prev_hbm, out_hbm, idx_v, rows_v, acc_sh, sem):
        cid = lax.axis_index("c")
        sid = lax.axis_index("s")
        wid = sid * NC + cid
        nv = (n_groups - wid + NW - 1) // NW

        def prefetch(t, slot):
            base = (wid + NW * t) * G
            pltpu.sync_copy(dst_hbm.at[pl.ds(base, G)], idx_v.at[slot])
            pltpu.async_copy(msg_hbm.at[pl.ds(base, G)], rows_v.at[slot],
                             sem.at[slot])

        pltpu.sync_copy(prev_hbm.at[cid, pl.ds(sid * rpt, rpt)],
                        acc_sh.at[pl.ds(sid * rpt, rpt)])
        plsc.subcore_barrier()

        @pl.when(nv > 0)
        def _():
            prefetch(0, 0)

        def body(t, carry):
            p = lax.rem(t, 2)

            @pl.when(t + 1 < nv)
            def _():
                prefetch(t + 1, 1 - p)

            @pl.when(t < nv)
            def _():
                base = (wid + NW * t) * G
                pltpu.make_async_copy(msg_hbm.at[pl.ds(base, G)],
                                      rows_v.at[p], sem.at[p]).wait()
                pltpu.sync_copy(rows_v.at[p], acc_sh.at[idx_v.at[p]],
                                add=True)

            return carry

        lax.fori_loop(0, n_iters, body, 0)
        plsc.subcore_barrier()
        pltpu.sync_copy(acc_sh.at[pl.ds(sid * rpt, rpt)],
                        out_hbm.at[cid, pl.ds(sid * rpt, rpt)])

    return k(msg, dst, prev)


def _tc_edge_chunk(xi, xj, ea_full, w0c, b0c, w1, b1, w2, b2, ln_g, ln_b,
                   gw1, gb1, eout_prev, blk_off):
    Ec = xi.shape[0]
    grid = Ec // BE

    def body(xi_ref, xj_ref, ea_ref, w0c_ref, b0c_ref, w1_ref, b1_ref,
             w2_ref, b2_ref, lng_ref, lnb_ref, gw1_ref, gb1_ref,
             *rest_refs):
        eout_ref, msg_ref = rest_refs[-2], rest_refs[-1]
        ea_blk = ea_ref[...]
        cat = jnp.concatenate([xi_ref[...], xj_ref[...], ea_blk], axis=1)
        h0 = jnp.dot(cat, w0c_ref[...], preferred_element_type=jnp.float32)
        h0 = jnp.maximum(h0 + b0c_ref[...], 0.0)
        h = h0[:, :H]
        gh = h0[:, H:]
        h = jnp.dot(h, w1_ref[...], preferred_element_type=jnp.float32)
        h = jnp.maximum(h + b1_ref[...], 0.0)
        h = jnp.dot(h, w2_ref[...], preferred_element_type=jnp.float32)
        h = h + b2_ref[...]
        m = jnp.mean(h, axis=1, keepdims=True)
        c = h - m
        v = jnp.mean(c * c, axis=1, keepdims=True)
        e_new = c * lax.rsqrt(v + 1e-5) * lng_ref[...] + lnb_ref[...]
        z = jnp.sum(gh * gw1_ref[...], axis=1, keepdims=True) + gb1_ref[0, 0]
        gate = 1.0 / (1.0 + jnp.exp(-z))
        eout_ref[...] = ea_blk + e_new
        msg_ref[...] = gate * e_new

    E = ea_full.shape[0]
    blk = pl.BlockSpec((BE, H), lambda i: (i, 0))
    off = pl.BlockSpec((BE, H), lambda i: (i + blk_off, 0))
    full = lambda shape: pl.BlockSpec(shape, lambda i: (0,) * len(shape))
    in_specs = [
        blk, blk, off,
        full((3 * H, 2 * H)), full((1, 2 * H)),
        full((H, H)), full((1, H)),
        full((H, H)), full((1, H)),
        full((1, H)), full((1, H)),
        full((1, H)), full((1, 1)),
    ]
    args = [xi, xj, ea_full, w0c, b0c, w1, b1, w2, b2, ln_g, ln_b, gw1, gb1]
    aliases = {}
    if eout_prev is not None:
        in_specs.append(pl.BlockSpec(memory_space=pl.ANY))
        args.append(eout_prev)
        aliases = {13: 0}

        def body_alias(*refs):
            body(*refs[:13], *refs[14:])

        body_fn = body_alias
    else:
        body_fn = body
    return pl.pallas_call(
        body_fn,
        grid=(grid,),
        in_specs=in_specs,
        out_specs=[off, blk],
        out_shape=[
            jax.ShapeDtypeStruct((E, H), jnp.float32),
            jax.ShapeDtypeStruct((Ec, H), jnp.float32),
        ],
        input_output_aliases=aliases,
    )(*args)


def _tc_node(x, a0, a1, w0, b0, w1, b1, w2, b2, ln_g, ln_b):
    N = x.shape[0]
    BN = 1000
    grid = N // BN

    def body(x_ref, a0_ref, a1_ref, w0_ref, b0_ref, w1_ref, b1_ref, w2_ref,
             b2_ref, lng_ref, lnb_ref, out_ref):
        x_blk = x_ref[...]
        aggr = a0_ref[...] + a1_ref[...]
        cat = jnp.concatenate([x_blk, aggr], axis=1)
        h = jnp.dot(cat, w0_ref[...], preferred_element_type=jnp.float32)
        h = jnp.maximum(h + b0_ref[...], 0.0)
        h = jnp.dot(h, w1_ref[...], preferred_element_type=jnp.float32)
        h = jnp.maximum(h + b1_ref[...], 0.0)
        h = jnp.dot(h, w2_ref[...], preferred_element_type=jnp.float32)
        h = h + b2_ref[...]
        m = jnp.mean(h, axis=1, keepdims=True)
        c = h - m
        v = jnp.mean(c * c, axis=1, keepdims=True)
        out_ref[...] = x_blk + (c * lax.rsqrt(v + 1e-5) * lng_ref[...]
                                + lnb_ref[...])

    blk = pl.BlockSpec((BN, H), lambda i: (i, 0))
    full = lambda shape: pl.BlockSpec(shape, lambda i: (0,) * len(shape))
    return pl.pallas_call(
        body,
        grid=(grid,),
        in_specs=[
            blk, blk, blk,
            full((2 * H, H)), full((1, H)),
            full((H, H)), full((1, H)),
            full((H, H)), full((1, H)),
            full((1, H)), full((1, H)),
        ],
        out_specs=blk,
        out_shape=jax.ShapeDtypeStruct((N, H), jnp.float32),
    )(x, a0, a1, w0, b0, w1, b1, w2, b2, ln_g, ln_b)


def kernel(x, edge_index, edge_attr, params):
    p = params
    src = edge_index[0]
    dst = edge_index[1]
    N = x.shape[0]
    E = edge_attr.shape[0]
    Ec = E // NCHUNK

    r1 = lambda a: a.reshape(1, H)
    w0c = jnp.concatenate([p['e_w0'], p['g_w0']], axis=1)
    b0c = jnp.concatenate([p['e_b0'], p['g_b0']]).reshape(1, 2 * H)

    x_pad = jnp.concatenate(
        [x, jnp.zeros((NPAD - N, H), jnp.float32)], axis=0)
    gathered = []
    for k in range(NCHUNK):
        sl = slice(k * Ec, (k + 1) * Ec)
        gathered.append((_sc_gather(x_pad, dst[sl], src[sl]), dst[sl]))

    eout = None
    msgs = []
    for k, ((xi, xj), _) in enumerate(gathered):
        eout, msg = _tc_edge_chunk(
            xi, xj, edge_attr, w0c, b0c, p['e_w1'], r1(p['e_b1']),
            p['e_w2'], r1(p['e_b2']), r1(p['e_ln_g']), r1(p['e_ln_b']),
            p['g_w1'].reshape(1, H), p['g_b1'].reshape(1, 1),
            eout, k * (Ec // BE))
        msgs.append(msg)

    parts = jnp.zeros((NC, NPAD, H), jnp.float32)
    for k, (_, dst_k) in enumerate(gathered):
        parts = _sc_scatter(msgs[k], dst_k, parts)

    x_new = _tc_node(
        x, parts[0, :N], parts[1, :N],
        p['n_w0'], r1(p['n_b0']), p['n_w1'], r1(p['n_b1']),
        p['n_w2'], r1(p['n_b2']), r1(p['n_ln_g']), r1(p['n_ln_b']))

    return (x_new, eout)


# speedup vs baseline: 2.0670x; 1.1054x over previous
import functools

import jax
import jax.numpy as jnp
from jax import lax
from jax.experimental import pallas as pl
from jax.experimental.pallas import tpu as pltpu
from jax.experimental.pallas import tpu_sc as plsc

H = 128
G = 128
NC = 2
NS = 16
NW = NC * NS
NPAD = 10240
NCHUNK = 5
BE = 1280


def _sc_mesh():
    return plsc.VectorSubcoreMesh(core_axis_name="c", subcore_axis_name="s")


def _sc_gather(x_pad, dst, src):
    GG = 80
    E = dst.shape[0]
    epw = E // NW
    nv = epw // GG
    rpt = NPAD // NS

    @functools.partial(
        pl.kernel,
        mesh=_sc_mesh(),
        out_type=(
            jax.ShapeDtypeStruct((E, H), jnp.float32),
            jax.ShapeDtypeStruct((E, H), jnp.float32),
        ),
        scratch_types=[
            pltpu.VMEM((epw,), jnp.int32),
            pltpu.VMEM((epw,), jnp.int32),
            pltpu.VMEM((2, GG, H), jnp.float32),
            pltpu.VMEM((2, GG, H), jnp.float32),
            pltpu.VMEM_SHARED((NPAD, H), jnp.float32),
            pltpu.SemaphoreType.DMA((2,)),
            pltpu.SemaphoreType.DMA((2,)),
            pltpu.SemaphoreType.DMA((2,)),
            pltpu.SemaphoreType.DMA((2,)),
        ],
    )
    def k(x_hbm, dst_hbm, src_hbm, xi_hbm, xj_hbm,
          idx_d, idx_s, rows_d, rows_s, x_sh,
          sem_gd, sem_gs, sem_sd, sem_ss):
        sid = lax.axis_index("s")
        wid = sid * NC + lax.axis_index("c")
        w0 = wid * epw

        pltpu.sync_copy(x_hbm.at[pl.ds(sid * rpt, rpt)],
                        x_sh.at[pl.ds(sid * rpt, rpt)])
        pltpu.sync_copy(dst_hbm.at[pl.ds(w0, epw)], idx_d)
        pltpu.sync_copy(src_hbm.at[pl.ds(w0, epw)], idx_s)
        plsc.subcore_barrier()

        def gather(t, p):
            pltpu.async_copy(x_sh.at[idx_d.at[pl.ds(t * GG, GG)]],
                             rows_d.at[p], sem_gd.at[p])
            pltpu.async_copy(x_sh.at[idx_s.at[pl.ds(t * GG, GG)]],
                             rows_s.at[p], sem_gs.at[p])

        def wait_gather(t, p):
            pltpu.make_async_copy(x_sh.at[idx_d.at[pl.ds(t * GG, GG)]],
                                  rows_d.at[p], sem_gd.at[p]).wait()
            pltpu.make_async_copy(x_sh.at[idx_s.at[pl.ds(t * GG, GG)]],
                                  rows_s.at[p], sem_gs.at[p]).wait()

        def store(t, p):
            base = w0 + t * GG
            pltpu.async_copy(rows_d.at[p], xi_hbm.at[pl.ds(base, GG)],
                             sem_sd.at[p])
            pltpu.async_copy(rows_s.at[p], xj_hbm.at[pl.ds(base, GG)],
                             sem_ss.at[p])

        def wait_store(t, p):
            base = w0 + t * GG
            pltpu.make_async_copy(rows_d.at[p], xi_hbm.at[pl.ds(base, GG)],
                                  sem_sd.at[p]).wait()
            pltpu.make_async_copy(rows_s.at[p], xj_hbm.at[pl.ds(base, GG)],
                                  sem_ss.at[p]).wait()

        gather(0, 0)

        def body(t, carry):
            p = lax.rem(t, 2)
            wait_gather(t, p)
            store(t, p)

            @pl.when(t + 1 < nv)
            def _():
                @pl.when(t >= 1)
                def _():
                    wait_store(t - 1, 1 - p)

                gather(t + 1, 1 - p)

            return carry

        lax.fori_loop(0, nv, body, 0)
        wait_store(nv - 2, lax.rem(nv - 2, 2))
        wait_store(nv - 1, lax.rem(nv - 1, 2))

    return k(x_pad, dst, src)


def _sc_scatter(msg, dst, prev):
    E = msg.shape[0]
    n_groups = E // G
    n_iters = (n_groups + NW - 1) // NW
    rpt = NPAD // NS

    @functools.partial(
        pl.kernel,
        mesh=_sc_mesh(),
        out_type=jax.ShapeDtypeStruct((NC, NPAD, H), jnp.float32),
        scratch_types=[
            pltpu.VMEM((2, G), jnp.int32),
            pltpu.VMEM((2, G, H), jnp.float32),
            pltpu.VMEM_SHARED((NPAD, H), jnp.float32),
            pltpu.SemaphoreType.DMA((2,)),
        ],
    )
    def k(msg_hbm, dst_hbm, prev_hbm, out_hbm, idx_v, rows_v, acc_sh, sem):
        cid = lax.axis_index("c")
        sid = lax.axis_index("s")
        wid = sid * NC + cid
        nv = (n_groups - wid + NW - 1) // NW

        def prefetch(t, slot):
            base = (wid + NW * t) * G
            pltpu.sync_copy(dst_hbm.at[pl.ds(base, G)], idx_v.at[slot])
            pltpu.async_copy(msg_hbm.at[pl.ds(base, G)], rows_v.at[slot],
                             sem.at[slot])

        pltpu.sync_copy(prev_hbm.at[cid, pl.ds(sid * rpt, rpt)],
                        acc_sh.at[pl.ds(sid * rpt, rpt)])
        plsc.subcore_barrier()

        @pl.when(nv > 0)
        def _():
            prefetch(0, 0)

        def body(t, carry):
            p = lax.rem(t, 2)

            @pl.when(t + 1 < nv)
            def _():
                prefetch(t + 1, 1 - p)

            @pl.when(t < nv)
            def _():
                base = (wid + NW * t) * G
                pltpu.make_async_copy(msg_hbm.at[pl.ds(base, G)],
                                      rows_v.at[p], sem.at[p]).wait()
                pltpu.sync_copy(rows_v.at[p], acc_sh.at[idx_v.at[p]],
                                add=True)

            return carry

        lax.fori_loop(0, n_iters, body, 0)
        plsc.subcore_barrier()
        pltpu.sync_copy(acc_sh.at[pl.ds(sid * rpt, rpt)],
                        out_hbm.at[cid, pl.ds(sid * rpt, rpt)])

    return k(msg, dst, prev)


def _tc_edge_chunk(xi, xj, ea_full, w0c, b0c, w1, b1, w2, b2, ln_g, ln_b,
                   gw1, gb1, eout_prev, blk_off):
    Ec = xi.shape[0]
    grid = Ec // BE

    def body(xi_ref, xj_ref, ea_ref, w0c_ref, b0c_ref, w1_ref, b1_ref,
             w2_ref, b2_ref, lng_ref, lnb_ref, gw1_ref, gb1_ref,
             *rest_refs):
        eout_ref, msg_ref = rest_refs[-2], rest_refs[-1]
        ea_blk = ea_ref[...]
        cat = jnp.concatenate([xi_ref[...], xj_ref[...], ea_blk], axis=1)
        h0 = jnp.dot(cat, w0c_ref[...], preferred_element_type=jnp.float32)
        h0 = jnp.maximum(h0 + b0c_ref[...], 0.0)
        h = h0[:, :H]
        gh = h0[:, H:]
        h = jnp.dot(h, w1_ref[...], preferred_element_type=jnp.float32)
        h = jnp.maximum(h + b1_ref[...], 0.0)
        h = jnp.dot(h, w2_ref[...], preferred_element_type=jnp.float32)
        h = h + b2_ref[...]
        m = jnp.mean(h, axis=1, keepdims=True)
        c = h - m
        v = jnp.mean(c * c, axis=1, keepdims=True)
        e_new = c * lax.rsqrt(v + 1e-5) * lng_ref[...] + lnb_ref[...]
        z = jnp.sum(gh * gw1_ref[...], axis=1, keepdims=True) + gb1_ref[0, 0]
        gate = 1.0 / (1.0 + jnp.exp(-z))
        eout_ref[...] = ea_blk + e_new
        msg_ref[...] = gate * e_new

    E = ea_full.shape[0]
    blk = pl.BlockSpec((BE, H), lambda i: (i, 0))
    off = pl.BlockSpec((BE, H), lambda i: (i + blk_off, 0))
    full = lambda shape: pl.BlockSpec(shape, lambda i: (0,) * len(shape))
    in_specs = [
        blk, blk, off,
        full((3 * H, 2 * H)), full((1, 2 * H)),
        full((H, H)), full((1, H)),
        full((H, H)), full((1, H)),
        full((1, H)), full((1, H)),
        full((1, H)), full((1, 1)),
    ]
    args = [xi, xj, ea_full, w0c, b0c, w1, b1, w2, b2, ln_g, ln_b, gw1, gb1]
    aliases = {}
    if eout_prev is not None:
        in_specs.append(pl.BlockSpec(memory_space=pl.ANY))
        args.append(eout_prev)
        aliases = {13: 0}

        def body_alias(*refs):
            body(*refs[:13], *refs[14:])

        body_fn = body_alias
    else:
        body_fn = body
    return pl.pallas_call(
        body_fn,
        grid=(grid,),
        in_specs=in_specs,
        out_specs=[off, blk],
        out_shape=[
            jax.ShapeDtypeStruct((E, H), jnp.float32),
            jax.ShapeDtypeStruct((Ec, H), jnp.float32),
        ],
        input_output_aliases=aliases,
    )(*args)


def _tc_node(x, a0, a1, w0, b0, w1, b1, w2, b2, ln_g, ln_b):
    N = x.shape[0]
    BN = 1000
    grid = N // BN

    def body(x_ref, a0_ref, a1_ref, w0_ref, b0_ref, w1_ref, b1_ref, w2_ref,
             b2_ref, lng_ref, lnb_ref, out_ref):
        x_blk = x_ref[...]
        aggr = a0_ref[...] + a1_ref[...]
        cat = jnp.concatenate([x_blk, aggr], axis=1)
        h = jnp.dot(cat, w0_ref[...], preferred_element_type=jnp.float32)
        h = jnp.maximum(h + b0_ref[...], 0.0)
        h = jnp.dot(h, w1_ref[...], preferred_element_type=jnp.float32)
        h = jnp.maximum(h + b1_ref[...], 0.0)
        h = jnp.dot(h, w2_ref[...], preferred_element_type=jnp.float32)
        h = h + b2_ref[...]
        m = jnp.mean(h, axis=1, keepdims=True)
        c = h - m
        v = jnp.mean(c * c, axis=1, keepdims=True)
        out_ref[...] = x_blk + (c * lax.rsqrt(v + 1e-5) * lng_ref[...]
                                + lnb_ref[...])

    blk = pl.BlockSpec((BN, H), lambda i: (i, 0))
    full = lambda shape: pl.BlockSpec(shape, lambda i: (0,) * len(shape))
    return pl.pallas_call(
        body,
        grid=(grid,),
        in_specs=[
            blk, blk, blk,
            full((2 * H, H)), full((1, H)),
            full((H, H)), full((1, H)),
            full((H, H)), full((1, H)),
            full((1, H)), full((1, H)),
        ],
        out_specs=blk,
        out_shape=jax.ShapeDtypeStruct((N, H), jnp.float32),
    )(x, a0, a1, w0, b0, w1, b1, w2, b2, ln_g, ln_b)


def kernel(x, edge_index, edge_attr, params):
    p = params
    src = edge_index[0]
    dst = edge_index[1]
    N = x.shape[0]
    E = edge_attr.shape[0]
    Ec = E // NCHUNK

    r1 = lambda a: a.reshape(1, H)
    w0c = jnp.concatenate([p['e_w0'], p['g_w0']], axis=1)
    b0c = jnp.concatenate([p['e_b0'], p['g_b0']]).reshape(1, 2 * H)

    x_pad = jnp.concatenate(
        [x, jnp.zeros((NPAD - N, H), jnp.float32)], axis=0)
    gathered = []
    for k in range(NCHUNK):
        sl = slice(k * Ec, (k + 1) * Ec)
        gathered.append((_sc_gather(x_pad, dst[sl], src[sl]), dst[sl]))

    eout = None
    msgs = []
    for k, ((xi, xj), _) in enumerate(gathered):
        eout, msg = _tc_edge_chunk(
            xi, xj, edge_attr, w0c, b0c, p['e_w1'], r1(p['e_b1']),
            p['e_w2'], r1(p['e_b2']), r1(p['e_ln_g']), r1(p['e_ln_b']),
            p['g_w1'].reshape(1, H), p['g_b1'].reshape(1, 1),
            eout, k * (Ec // BE))
        msgs.append(msg)

    parts = jnp.zeros((NC, NPAD, H), jnp.float32)
    for k, (_, dst_k) in enumerate(gathered):
        parts = _sc_scatter(msgs[k], dst_k, parts)

    x_new = _tc_node(
        x, parts[0, :N], parts[1, :N],
        p['n_w0'], r1(p['n_b0']), p['n_w1'], r1(p['n_b1']),
        p['n_w2'], r1(p['n_b2']), r1(p['n_ln_g']), r1(p['n_ln_b']))

    return (x_new, eout)


# speedup vs baseline: 2.0692x; 1.0011x over previous
import functools

import jax
import jax.numpy as jnp
from jax import lax
from jax.experimental import pallas as pl
from jax.experimental.pallas import tpu as pltpu
from jax.experimental.pallas import tpu_sc as plsc

H = 128
G = 128
NC = 2
NS = 16
NW = NC * NS
NPAD = 10240
NCHUNK = 5
BE = 1280


def _sc_mesh():
    return plsc.VectorSubcoreMesh(core_axis_name="c", subcore_axis_name="s")


def _sc_gather(x_pad, dst, src):
    GG = 80
    E = dst.shape[0]
    epw = E // NW
    nv = epw // GG
    rpt = NPAD // NS

    @functools.partial(
        pl.kernel,
        mesh=_sc_mesh(),
        out_type=(
            jax.ShapeDtypeStruct((E, H), jnp.float32),
            jax.ShapeDtypeStruct((E, H), jnp.float32),
        ),
        scratch_types=[
            pltpu.VMEM((epw,), jnp.int32),
            pltpu.VMEM((epw,), jnp.int32),
            pltpu.VMEM((2, GG, H), jnp.float32),
            pltpu.VMEM((2, GG, H), jnp.float32),
            pltpu.VMEM_SHARED((NPAD, H), jnp.float32),
            pltpu.SemaphoreType.DMA((2,)),
            pltpu.SemaphoreType.DMA((2,)),
            pltpu.SemaphoreType.DMA((2,)),
            pltpu.SemaphoreType.DMA((2,)),
        ],
    )
    def k(x_hbm, dst_hbm, src_hbm, xi_hbm, xj_hbm,
          idx_d, idx_s, rows_d, rows_s, x_sh,
          sem_gd, sem_gs, sem_sd, sem_ss):
        sid = lax.axis_index("s")
        wid = sid * NC + lax.axis_index("c")
        w0 = wid * epw

        pltpu.sync_copy(x_hbm.at[pl.ds(sid * rpt, rpt)],
                        x_sh.at[pl.ds(sid * rpt, rpt)])
        pltpu.sync_copy(dst_hbm.at[pl.ds(w0, epw)], idx_d)
        pltpu.sync_copy(src_hbm.at[pl.ds(w0, epw)], idx_s)
        plsc.subcore_barrier()

        def gather(t, p):
            pltpu.async_copy(x_sh.at[idx_d.at[pl.ds(t * GG, GG)]],
                             rows_d.at[p], sem_gd.at[p])
            pltpu.async_copy(x_sh.at[idx_s.at[pl.ds(t * GG, GG)]],
                             rows_s.at[p], sem_gs.at[p])

        def wait_gather(t, p):
            pltpu.make_async_copy(x_sh.at[idx_d.at[pl.ds(t * GG, GG)]],
                                  rows_d.at[p], sem_gd.at[p]).wait()
            pltpu.make_async_copy(x_sh.at[idx_s.at[pl.ds(t * GG, GG)]],
                                  rows_s.at[p], sem_gs.at[p]).wait()

        def store(t, p):
            base = w0 + t * GG
            pltpu.async_copy(rows_d.at[p], xi_hbm.at[pl.ds(base, GG)],
                             sem_sd.at[p])
            pltpu.async_copy(rows_s.at[p], xj_hbm.at[pl.ds(base, GG)],
                             sem_ss.at[p])

        def wait_store(t, p):
            base = w0 + t * GG
            pltpu.make_async_copy(rows_d.at[p], xi_hbm.at[pl.ds(base, GG)],
                                  sem_sd.at[p]).wait()
            pltpu.make_async_copy(rows_s.at[p], xj_hbm.at[pl.ds(base, GG)],
                                  sem_ss.at[p]).wait()

        gather(0, 0)

        def body(t, carry):
            p = lax.rem(t, 2)
            wait_gather(t, p)
            store(t, p)

            @pl.when(t + 1 < nv)
            def _():
                @pl.when(t >= 1)
                def _():
                    wait_store(t - 1, 1 - p)

                gather(t + 1, 1 - p)

            return carry

        lax.fori_loop(0, nv, body, 0)
        wait_store(nv - 2, lax.rem(nv - 2, 2))
        wait_store(nv - 1, lax.rem(nv - 1, 2))

    return k(x_pad, dst, src)


def _sc_scatter(msg, dst, prev):
    GS = 80
    E = msg.shape[0]
    epw = E // NW
    nv = epw // GS
    rpt = NPAD // NS
    dst2 = dst.reshape(NW, epw // GS, GS)

    @functools.partial(
        pl.kernel,
        mesh=_sc_mesh(),
        out_type=jax.ShapeDtypeStruct((NC, NPAD, H), jnp.float32),
        scratch_types=[
            pltpu.VMEM((epw // GS, GS), jnp.int32),
            pltpu.VMEM((2, GS, H), jnp.float32),
            pltpu.VMEM_SHARED((NPAD, H), jnp.float32),
            pltpu.SemaphoreType.DMA((2,)),
        ],
    )
    def k(msg_hbm, dst_hbm, prev_hbm, out_hbm, idx_v, rows_v, acc_sh, sem):
        cid = lax.axis_index("c")
        sid = lax.axis_index("s")
        wid = sid * NC + cid
        w0 = wid * epw

        def prefetch(t, slot):
            pltpu.async_copy(msg_hbm.at[pl.ds(w0 + t * GS, GS)],
                             rows_v.at[slot], sem.at[slot])

        pltpu.sync_copy(prev_hbm.at[cid, pl.ds(sid * rpt, rpt)],
                        acc_sh.at[pl.ds(sid * rpt, rpt)])
        pltpu.sync_copy(dst_hbm.at[wid], idx_v)
        plsc.subcore_barrier()
        prefetch(0, 0)

        def body(t, carry):
            p = lax.rem(t, 2)

            @pl.when(t + 1 < nv)
            def _():
                prefetch(t + 1, 1 - p)

            pltpu.make_async_copy(msg_hbm.at[pl.ds(w0 + t * GS, GS)],
                                  rows_v.at[p], sem.at[p]).wait()
            pltpu.sync_copy(rows_v.at[p], acc_sh.at[idx_v.at[t]],
                            add=True)

            return carry

        lax.fori_loop(0, nv, body, 0)
        plsc.subcore_barrier()
        pltpu.sync_copy(acc_sh.at[pl.ds(sid * rpt, rpt)],
                        out_hbm.at[cid, pl.ds(sid * rpt, rpt)])

    return k(msg, dst2, prev)


def _tc_edge_chunk(xi, xj, ea_full, w0c, b0c, w1, b1, w2, b2, ln_g, ln_b,
                   gw1, gb1, eout_prev, blk_off):
    Ec = xi.shape[0]
    grid = Ec // BE

    def body(xi_ref, xj_ref, ea_ref, w0c_ref, b0c_ref, w1_ref, b1_ref,
             w2_ref, b2_ref, lng_ref, lnb_ref, gw1_ref, gb1_ref,
             *rest_refs):
        eout_ref, msg_ref = rest_refs[-2], rest_refs[-1]
        ea_blk = ea_ref[...]
        cat = jnp.concatenate([xi_ref[...], xj_ref[...], ea_blk], axis=1)
        h0 = jnp.dot(cat, w0c_ref[...], preferred_element_type=jnp.float32)
        h0 = jnp.maximum(h0 + b0c_ref[...], 0.0)
        h = h0[:, :H]
        gh = h0[:, H:]
        h = jnp.dot(h, w1_ref[...], preferred_element_type=jnp.float32)
        h = jnp.maximum(h + b1_ref[...], 0.0)
        h = jnp.dot(h, w2_ref[...], preferred_element_type=jnp.float32)
        h = h + b2_ref[...]
        m = jnp.mean(h, axis=1, keepdims=True)
        c = h - m
        v = jnp.mean(c * c, axis=1, keepdims=True)
        e_new = c * lax.rsqrt(v + 1e-5) * lng_ref[...] + lnb_ref[...]
        z = jnp.sum(gh * gw1_ref[...], axis=1, keepdims=True) + gb1_ref[0, 0]
        gate = 1.0 / (1.0 + jnp.exp(-z))
        eout_ref[...] = ea_blk + e_new
        msg_ref[...] = gate * e_new

    E = ea_full.shape[0]
    blk = pl.BlockSpec((BE, H), lambda i: (i, 0))
    off = pl.BlockSpec((BE, H), lambda i: (i + blk_off, 0))
    full = lambda shape: pl.BlockSpec(shape, lambda i: (0,) * len(shape))
    in_specs = [
        blk, blk, off,
        full((3 * H, 2 * H)), full((1, 2 * H)),
        full((H, H)), full((1, H)),
        full((H, H)), full((1, H)),
        full((1, H)), full((1, H)),
        full((1, H)), full((1, 1)),
    ]
    args = [xi, xj, ea_full, w0c, b0c, w1, b1, w2, b2, ln_g, ln_b, gw1, gb1]
    aliases = {}
    if eout_prev is not None:
        in_specs.append(pl.BlockSpec(memory_space=pl.ANY))
        args.append(eout_prev)
        aliases = {13: 0}

        def body_alias(*refs):
            body(*refs[:13], *refs[14:])

        body_fn = body_alias
    else:
        body_fn = body
    return pl.pallas_call(
        body_fn,
        grid=(grid,),
        in_specs=in_specs,
        out_specs=[off, blk],
        out_shape=[
            jax.ShapeDtypeStruct((E, H), jnp.float32),
            jax.ShapeDtypeStruct((Ec, H), jnp.float32),
        ],
        input_output_aliases=aliases,
    )(*args)


def _tc_node(x, a0, a1, w0, b0, w1, b1, w2, b2, ln_g, ln_b):
    N = x.shape[0]
    BN = 1000
    grid = N // BN

    def body(x_ref, a0_ref, a1_ref, w0_ref, b0_ref, w1_ref, b1_ref, w2_ref,
             b2_ref, lng_ref, lnb_ref, out_ref):
        x_blk = x_ref[...]
        aggr = a0_ref[...] + a1_ref[...]
        cat = jnp.concatenate([x_blk, aggr], axis=1)
        h = jnp.dot(cat, w0_ref[...], preferred_element_type=jnp.float32)
        h = jnp.maximum(h + b0_ref[...], 0.0)
        h = jnp.dot(h, w1_ref[...], preferred_element_type=jnp.float32)
        h = jnp.maximum(h + b1_ref[...], 0.0)
        h = jnp.dot(h, w2_ref[...], preferred_element_type=jnp.float32)
        h = h + b2_ref[...]
        m = jnp.mean(h, axis=1, keepdims=True)
        c = h - m
        v = jnp.mean(c * c, axis=1, keepdims=True)
        out_ref[...] = x_blk + (c * lax.rsqrt(v + 1e-5) * lng_ref[...]
                                + lnb_ref[...])

    blk = pl.BlockSpec((BN, H), lambda i: (i, 0))
    full = lambda shape: pl.BlockSpec(shape, lambda i: (0,) * len(shape))
    return pl.pallas_call(
        body,
        grid=(grid,),
        in_specs=[
            blk, blk, blk,
            full((2 * H, H)), full((1, H)),
            full((H, H)), full((1, H)),
            full((H, H)), full((1, H)),
            full((1, H)), full((1, H)),
        ],
        out_specs=blk,
        out_shape=jax.ShapeDtypeStruct((N, H), jnp.float32),
    )(x, a0, a1, w0, b0, w1, b1, w2, b2, ln_g, ln_b)


def kernel(x, edge_index, edge_attr, params):
    p = params
    src = edge_index[0]
    dst = edge_index[1]
    N = x.shape[0]
    E = edge_attr.shape[0]
    Ec = E // NCHUNK

    r1 = lambda a: a.reshape(1, H)
    w0c = jnp.concatenate([p['e_w0'], p['g_w0']], axis=1)
    b0c = jnp.concatenate([p['e_b0'], p['g_b0']]).reshape(1, 2 * H)

    x_pad = jnp.concatenate(
        [x, jnp.zeros((NPAD - N, H), jnp.float32)], axis=0)
    gathered = []
    for k in range(NCHUNK):
        sl = slice(k * Ec, (k + 1) * Ec)
        gathered.append((_sc_gather(x_pad, dst[sl], src[sl]), dst[sl]))

    eout = None
    msgs = []
    for k, ((xi, xj), _) in enumerate(gathered):
        eout, msg = _tc_edge_chunk(
            xi, xj, edge_attr, w0c, b0c, p['e_w1'], r1(p['e_b1']),
            p['e_w2'], r1(p['e_b2']), r1(p['e_ln_g']), r1(p['e_ln_b']),
            p['g_w1'].reshape(1, H), p['g_b1'].reshape(1, 1),
            eout, k * (Ec // BE))
        msgs.append(msg)

    parts = jnp.zeros((NC, NPAD, H), jnp.float32)
    for k, (_, dst_k) in enumerate(gathered):
        parts = _sc_scatter(msgs[k], dst_k, parts)

    x_new = _tc_node(
        x, parts[0, :N], parts[1, :N],
        p['n_w0'], r1(p['n_b0']), p['n_w1'], r1(p['n_b1']),
        p['n_w2'], r1(p['n_b2']), r1(p['n_ln_g']), r1(p['n_ln_b']))

    return (x_new, eout)
